# Initial kernel scaffold; baseline (speedup 1.0000x reference)
import jax
import jax.numpy as jnp
from jax.experimental import pallas as pl


def kernel(hidden_states, gate_up_proj, down_proj, router_w, shared_gate_w, shared_up_w, shared_down_w):
    raise NotImplementedError("write your pallas kernel here")


# speedup vs baseline: 2.0607x; 2.0607x over previous
import functools

import jax
import jax.numpy as jnp
from jax import lax
from jax.experimental import pallas as pl
from jax.experimental.pallas import tpu as pltpu
from jax.experimental.pallas import tpu_sc as plsc

E = 8
H = 1024
I2 = 2048
T = 4096
BM = 256
NV = 23
TPAD = NV * BM
LN = 128
RS = H // LN
GW = 128


def _cumsum_lanes(x, steps=(1, 2, 4, 8, 16, 32, 64)):
    n = x.shape[1]
    for k in steps:
        pad = jnp.zeros((x.shape[0], k), x.dtype)
        x = x + jnp.concatenate([pad, x[:, : n - k]], axis=1)
    return x


def _cumsum_rows(x):
    n = x.shape[0]
    k = 1
    while k < n:
        pad = jnp.zeros((k, x.shape[1]), x.dtype)
        x = x + jnp.concatenate([pad, x[: n - k]], axis=0)
        k *= 2
    return x


def _router_body(hs_ref, rwt_ref, hsc_ref, pos_ref, md_ref):
    hs = hs_ref[...]
    logits = jnp.dot(hs, rwt_ref[...], preferred_element_type=jnp.float32)
    lane = lax.broadcasted_iota(jnp.int32, (T, LN), 1)
    logits = jnp.where(lane < E, logits, jnp.float32(-1e30))

    mx = jnp.max(logits, axis=1, keepdims=True)
    eq = (logits == mx).astype(jnp.int32)
    first = _cumsum_lanes(eq)
    onehot = jnp.where((eq == 1) & (first == 1), 1, 0)

    score = 1.0 / (1.0 + jnp.exp(-mx))
    hsc_ref[...] = hs * score

    ranks = _cumsum_rows(onehot)
    counts = ranks[T - 1 : T, :]
    ptiles = lax.shift_right_logical(counts + (BM - 1), 8)
    s_incl = _cumsum_lanes(ptiles)
    s_excl = s_incl - ptiles
    n_act = s_incl[0:1, LN - 1 : LN]
    poff = s_excl * BM

    pos = (
        jnp.sum(onehot * jnp.broadcast_to(poff, (T, LN)), axis=1, keepdims=True)
        + jnp.sum(onehot * ranks, axis=1, keepdims=True)
        - 1
    )
    sub = lax.broadcasted_iota(jnp.int32, (T, LN), 1)
    pos_ref[...] = (pos * RS + sub)[:, :RS]

    lane_r = lax.broadcasted_iota(jnp.int32, (1, LN), 1)
    veff_row = jnp.minimum(lane_r, n_act - 1)
    actv_row = (lane_r < n_act).astype(jnp.int32)

    w_sub = lax.broadcasted_iota(jnp.int32, (LN, LN), 0)
    lane2 = lax.broadcasted_iota(jnp.int32, (LN, LN), 1)
    veff_sub = jnp.minimum(w_sub, n_act - 1)
    ind = jnp.where(
        (jnp.broadcast_to(s_excl, (LN, LN)) <= veff_sub) & (lane2 < E), 1, 0
    )
    exp_col = jnp.sum(ind, axis=1, keepdims=True) - 1
    ident = (w_sub == lane2).astype(jnp.int32)
    exp_row = jnp.sum(
        jnp.broadcast_to(exp_col, (LN, LN)) * ident, axis=0, keepdims=True
    )
    md_ref[...] = jnp.concatenate(
        [veff_row, exp_row, actv_row, jnp.zeros((5, LN), jnp.int32)], axis=0
    )


def _run_router(hs, rwt, *, interpret=False):
    return pl.pallas_call(
        _router_body,
        out_shape=[
            jax.ShapeDtypeStruct((T, H), jnp.float32),
            jax.ShapeDtypeStruct((T, RS), jnp.int32),
            jax.ShapeDtypeStruct((8, LN), jnp.int32),
        ],
        interpret=interpret,
    )(hs, rwt)


def _gmm_body(ve_ref, ex_ref, ac_ref, x_ref, gw_ref, uw_ref, dw_ref, out_ref):
    g = pl.program_id(0)

    @pl.when(ac_ref[g] == 1)
    def _():
        xb = x_ref[...].astype(jnp.bfloat16)
        gg = jnp.dot(xb, gw_ref[0], preferred_element_type=jnp.float32)
        uu = jnp.dot(xb, uw_ref[0], preferred_element_type=jnp.float32)
        act = uu * (gg / (1.0 + jnp.exp(-gg)))
        out_ref[...] = jnp.dot(
            act.astype(jnp.bfloat16), dw_ref[0], preferred_element_type=jnp.float32
        )


def _run_gmm(ve, ex, ac, xs, gw, uw, dw, *, interpret=False):
    grid_spec = pltpu.PrefetchScalarGridSpec(
        num_scalar_prefetch=3,
        grid=(NV,),
        in_specs=[
            pl.BlockSpec((BM, H), lambda g, ve, ex, ac: (ve[g], 0)),
            pl.BlockSpec((1, H, I2), lambda g, ve, ex, ac: (ex[g], 0, 0)),
            pl.BlockSpec((1, H, I2), lambda g, ve, ex, ac: (ex[g], 0, 0)),
            pl.BlockSpec((1, I2, H), lambda g, ve, ex, ac: (ex[g], 0, 0)),
        ],
        out_specs=pl.BlockSpec((BM, H), lambda g, ve, ex, ac: (ve[g], 0)),
    )
    return pl.pallas_call(
        _gmm_body,
        grid_spec=grid_spec,
        out_shape=jax.ShapeDtypeStruct((TPAD, H), jnp.float32),
        interpret=interpret,
    )(ve, ex, ac, xs, gw, uw, dw)


def _shared_body(hs_ref, y_ref, gt_ref, ut_ref, dt_ref, o_ref):
    xb = hs_ref[...].astype(jnp.bfloat16)
    gg = jnp.dot(xb, gt_ref[...], preferred_element_type=jnp.float32)
    uu = jnp.dot(xb, ut_ref[...], preferred_element_type=jnp.float32)
    act = (gg / (1.0 + jnp.exp(-gg))) * uu
    o_ref[...] = (
        jnp.dot(act.astype(jnp.bfloat16), dt_ref[...], preferred_element_type=jnp.float32)
        + y_ref[...]
    )


def _run_shared(hs, y, gt, ut, dt, *, interpret=False):
    nb = T // BM
    return pl.pallas_call(
        _shared_body,
        grid=(nb,),
        in_specs=[
            pl.BlockSpec((BM, H), lambda i: (i, 0)),
            pl.BlockSpec((BM, H), lambda i: (i, 0)),
            pl.BlockSpec((H, I2), lambda i: (0, 0)),
            pl.BlockSpec((H, I2), lambda i: (0, 0)),
            pl.BlockSpec((I2, H), lambda i: (0, 0)),
        ],
        out_specs=pl.BlockSpec((BM, H), lambda i: (i, 0)),
        out_shape=jax.ShapeDtypeStruct((T, H), jnp.float32),
        interpret=interpret,
    )(hs, y, gt, ut, dt)


def _sc_mesh():
    return plsc.VectorSubcoreMesh(core_axis_name="core", subcore_axis_name="subcore")


def _sc_scatter_rows(rows, idx2d):
    nsub = T * RS

    @functools.partial(
        pl.kernel,
        out_type=jax.ShapeDtypeStruct((TPAD * RS, LN), jnp.float32),
        mesh=_sc_mesh(),
    )
    def k(x_hbm, i_hbm, o_hbm):
        def body(x_vmem, i_vmem):
            pltpu.sync_copy(x_vmem, o_hbm.at[i_vmem.at[0]])

        pltpu.emit_pipeline(
            body,
            grid=(nsub // GW,),
            in_specs=[
                pl.BlockSpec((GW, LN), lambda i: (i, 0)),
                pl.BlockSpec((1, GW), lambda i: (0, i)),
            ],
            out_specs=[],
            core_axis_name="subcore",
            dimension_semantics=(pltpu.PARALLEL,),
        )(x_hbm, i_hbm)

    return k(rows, idx2d)


def _sc_gather_rows(table, idx2d):
    nsub = T * RS

    @functools.partial(
        pl.kernel,
        out_type=jax.ShapeDtypeStruct((nsub, LN), jnp.float32),
        mesh=_sc_mesh(),
    )
    def k(y_hbm, i_hbm, o_hbm):
        def body(i_vmem, o_vmem):
            pltpu.sync_copy(y_hbm.at[i_vmem.at[0]], o_vmem)

        pltpu.emit_pipeline(
            body,
            grid=(nsub // GW,),
            in_specs=[pl.BlockSpec((1, GW), lambda i: (0, i))],
            out_specs=[pl.BlockSpec((GW, LN), lambda i: (i, 0))],
            core_axis_name="subcore",
            dimension_semantics=(pltpu.PARALLEL,),
        )(i_hbm, o_hbm)

    return k(table, idx2d)


def kernel(hidden_states, gate_up_proj, down_proj, router_w,
           shared_gate_w, shared_up_w, shared_down_w):
    hs = hidden_states.reshape(-1, H)
    rwt = jnp.pad(router_w.T, ((0, 0), (0, LN - E)))

    hsc, pos8, md = _run_router(hs, rwt)
    ve, ex, ac = md[0], md[1], md[2]
    pos2d = pos8.reshape(1, T * RS)

    xs = _sc_scatter_rows(hsc.reshape(T * RS, LN), pos2d)

    gw = gate_up_proj[:, :, :I2].astype(jnp.bfloat16)
    uw = gate_up_proj[:, :, I2:].astype(jnp.bfloat16)
    dw = down_proj.astype(jnp.bfloat16)
    ys = _run_gmm(ve, ex, ac, xs.reshape(TPAD, H), gw, uw, dw)

    y = _sc_gather_rows(ys.reshape(TPAD * RS, LN), pos2d)
    y = y.reshape(T, H)

    gt = shared_gate_w.T.astype(jnp.bfloat16)
    ut = shared_up_w.T.astype(jnp.bfloat16)
    dt = shared_down_w.T.astype(jnp.bfloat16)
    return _run_shared(hs, y, gt, ut, dt)


# speedup vs baseline: 2.6137x; 1.2684x over previous
import functools

import jax
import jax.numpy as jnp
from jax import lax
from jax.experimental import pallas as pl
from jax.experimental.pallas import tpu as pltpu
from jax.experimental.pallas import tpu_sc as plsc

E = 8
H = 1024
I2 = 2048
T = 4096
BM = 256
NV = 23
TPAD = NV * BM
LN = 128
RS = H // LN
GW = 128


def _cumsum_lanes(x, steps=(1, 2, 4, 8, 16, 32, 64)):
    n = x.shape[1]
    for k in steps:
        pad = jnp.zeros((x.shape[0], k), x.dtype)
        x = x + jnp.concatenate([pad, x[:, : n - k]], axis=1)
    return x


def _cumsum_rows(x):
    n = x.shape[0]
    k = 1
    while k < n:
        pad = jnp.zeros((k, x.shape[1]), x.dtype)
        x = x + jnp.concatenate([pad, x[: n - k]], axis=0)
        k *= 2
    return x


def _router_body(hs_ref, rwt_ref, hsc_ref, pos_ref, md_ref):
    hs = hs_ref[...]
    logits = jnp.dot(hs, rwt_ref[...], preferred_element_type=jnp.float32)
    lane = lax.broadcasted_iota(jnp.int32, (T, LN), 1)
    logits = jnp.where(lane < E, logits, jnp.float32(-1e30))

    mx = jnp.max(logits, axis=1, keepdims=True)
    eq = (logits == mx).astype(jnp.int32)
    first = _cumsum_lanes(eq)
    onehot = jnp.where((eq == 1) & (first == 1), 1, 0)

    score = 1.0 / (1.0 + jnp.exp(-mx))
    hsc_ref[...] = hs * score

    ranks = _cumsum_rows(onehot)
    counts = ranks[T - 1 : T, :]
    ptiles = lax.shift_right_logical(counts + (BM - 1), 8)
    s_incl = _cumsum_lanes(ptiles)
    s_excl = s_incl - ptiles
    n_act = s_incl[0:1, LN - 1 : LN]
    poff = s_excl * BM

    pos = (
        jnp.sum(onehot * jnp.broadcast_to(poff, (T, LN)), axis=1, keepdims=True)
        + jnp.sum(onehot * ranks, axis=1, keepdims=True)
        - 1
    )
    sub = lax.broadcasted_iota(jnp.int32, (T, LN), 1)
    pos_ref[...] = (pos * RS + sub)[:, :RS]

    lane_r = lax.broadcasted_iota(jnp.int32, (1, LN), 1)
    veff_row = jnp.minimum(lane_r, n_act - 1)
    actv_row = (lane_r < n_act).astype(jnp.int32)

    w_sub = lax.broadcasted_iota(jnp.int32, (LN, LN), 0)
    lane2 = lax.broadcasted_iota(jnp.int32, (LN, LN), 1)
    veff_sub = jnp.minimum(w_sub, n_act - 1)
    ind = jnp.where(
        (jnp.broadcast_to(s_excl, (LN, LN)) <= veff_sub) & (lane2 < E), 1, 0
    )
    exp_col = jnp.sum(ind, axis=1, keepdims=True) - 1
    ident = (w_sub == lane2).astype(jnp.int32)
    exp_row = jnp.sum(
        jnp.broadcast_to(exp_col, (LN, LN)) * ident, axis=0, keepdims=True
    )
    md_ref[...] = jnp.concatenate(
        [veff_row, exp_row, actv_row, jnp.zeros((5, LN), jnp.int32)], axis=0
    )


def _run_router(hs, rwt, *, interpret=False):
    return pl.pallas_call(
        _router_body,
        out_shape=[
            jax.ShapeDtypeStruct((T, H), jnp.float32),
            jax.ShapeDtypeStruct((T, RS), jnp.int32),
            jax.ShapeDtypeStruct((8, LN), jnp.int32),
        ],
        interpret=interpret,
    )(hs, rwt)


def _gmm_body(ve_ref, ex_ref, ac_ref, x_ref, gw_ref, uw_ref, dw_ref, out_ref):
    g = pl.program_id(0)

    @pl.when(ac_ref[g] == 1)
    def _():
        xb = x_ref[...].astype(jnp.bfloat16)
        gg = jnp.dot(
            xb, gw_ref[0].astype(jnp.bfloat16), preferred_element_type=jnp.float32
        )
        uu = jnp.dot(
            xb, uw_ref[0].astype(jnp.bfloat16), preferred_element_type=jnp.float32
        )
        act = uu * (gg / (1.0 + jnp.exp(-gg)))
        out_ref[...] = jnp.dot(
            act.astype(jnp.bfloat16),
            dw_ref[0].astype(jnp.bfloat16),
            preferred_element_type=jnp.float32,
        )


def _run_gmm(ve, ex, ac, xs, gw, uw, dw, *, interpret=False):
    grid_spec = pltpu.PrefetchScalarGridSpec(
        num_scalar_prefetch=3,
        grid=(NV,),
        in_specs=[
            pl.BlockSpec((BM, H), lambda g, ve, ex, ac: (ve[g], 0)),
            pl.BlockSpec((1, H, I2), lambda g, ve, ex, ac: (ex[g], 0, 0)),
            pl.BlockSpec((1, H, I2), lambda g, ve, ex, ac: (ex[g], 0, 1)),
            pl.BlockSpec((1, I2, H), lambda g, ve, ex, ac: (ex[g], 0, 0)),
        ],
        out_specs=pl.BlockSpec((BM, H), lambda g, ve, ex, ac: (ve[g], 0)),
    )
    return pl.pallas_call(
        _gmm_body,
        grid_spec=grid_spec,
        out_shape=jax.ShapeDtypeStruct((TPAD, H), jnp.float32),
        interpret=interpret,
    )(ve, ex, ac, xs, gw, uw, dw)


def _shared_body(hs_ref, y_ref, gt_ref, ut_ref, dt_ref, o_ref):
    xb = hs_ref[...].astype(jnp.bfloat16)
    gg = jnp.dot(xb, gt_ref[...], preferred_element_type=jnp.float32)
    uu = jnp.dot(xb, ut_ref[...], preferred_element_type=jnp.float32)
    act = (gg / (1.0 + jnp.exp(-gg))) * uu
    o_ref[...] = (
        jnp.dot(act.astype(jnp.bfloat16), dt_ref[...], preferred_element_type=jnp.float32)
        + y_ref[...]
    )


def _run_shared(hs, y, gt, ut, dt, *, interpret=False):
    nb = T // BM
    return pl.pallas_call(
        _shared_body,
        grid=(nb,),
        in_specs=[
            pl.BlockSpec((BM, H), lambda i: (i, 0)),
            pl.BlockSpec((BM, H), lambda i: (i, 0)),
            pl.BlockSpec((H, I2), lambda i: (0, 0)),
            pl.BlockSpec((H, I2), lambda i: (0, 0)),
            pl.BlockSpec((I2, H), lambda i: (0, 0)),
        ],
        out_specs=pl.BlockSpec((BM, H), lambda i: (i, 0)),
        out_shape=jax.ShapeDtypeStruct((T, H), jnp.float32),
        interpret=interpret,
    )(hs, y, gt, ut, dt)


def _sc_mesh():
    return plsc.VectorSubcoreMesh(core_axis_name="core", subcore_axis_name="subcore")


def _sc_scatter_rows(rows, idx2d):
    nsub = T * RS

    @functools.partial(
        pl.kernel,
        out_type=jax.ShapeDtypeStruct((TPAD * RS, LN), jnp.float32),
        mesh=_sc_mesh(),
    )
    def k(x_hbm, i_hbm, o_hbm):
        def body(x_vmem, i_vmem):
            pltpu.sync_copy(x_vmem, o_hbm.at[i_vmem.at[0]])

        pltpu.emit_pipeline(
            body,
            grid=(nsub // GW,),
            in_specs=[
                pl.BlockSpec((GW, LN), lambda i: (i, 0)),
                pl.BlockSpec((1, GW), lambda i: (0, i)),
            ],
            out_specs=[],
            core_axis_name="subcore",
            dimension_semantics=(pltpu.PARALLEL,),
        )(x_hbm, i_hbm)

    return k(rows, idx2d)


def _sc_gather_rows(table, idx2d):
    nsub = T * RS

    @functools.partial(
        pl.kernel,
        out_type=jax.ShapeDtypeStruct((nsub, LN), jnp.float32),
        mesh=_sc_mesh(),
    )
    def k(y_hbm, i_hbm, o_hbm):
        def body(i_vmem, o_vmem):
            pltpu.sync_copy(y_hbm.at[i_vmem.at[0]], o_vmem)

        pltpu.emit_pipeline(
            body,
            grid=(nsub // GW,),
            in_specs=[pl.BlockSpec((1, GW), lambda i: (0, i))],
            out_specs=[pl.BlockSpec((GW, LN), lambda i: (i, 0))],
            core_axis_name="subcore",
            dimension_semantics=(pltpu.PARALLEL,),
        )(i_hbm, o_hbm)

    return k(table, idx2d)


def kernel(hidden_states, gate_up_proj, down_proj, router_w,
           shared_gate_w, shared_up_w, shared_down_w):
    hs = hidden_states.reshape(-1, H)
    rwt = jnp.pad(router_w.T, ((0, 0), (0, LN - E)))

    hsc, pos8, md = _run_router(hs, rwt)
    ve, ex, ac = md[0], md[1], md[2]
    pos2d = pos8.reshape(1, T * RS)

    xs = _sc_scatter_rows(hsc.reshape(T * RS, LN), pos2d)

    ys = _run_gmm(
        ve, ex, ac, xs.reshape(TPAD, H), gate_up_proj, gate_up_proj, down_proj
    )

    y = _sc_gather_rows(ys.reshape(TPAD * RS, LN), pos2d)
    y = y.reshape(T, H)

    gt = shared_gate_w.T.astype(jnp.bfloat16)
    ut = shared_up_w.T.astype(jnp.bfloat16)
    dt = shared_down_w.T.astype(jnp.bfloat16)
    return _run_shared(hs, y, gt, ut, dt)


# speedup vs baseline: 3.3639x; 1.2870x over previous
import functools

import jax
import jax.numpy as jnp
from jax import lax
from jax.experimental import pallas as pl
from jax.experimental.pallas import tpu as pltpu
from jax.experimental.pallas import tpu_sc as plsc

E = 8
H = 1024
I2 = 2048
T = 4096
BM = 256
NV = 23
TPAD = NV * BM
LN = 128
RS = H // LN
GW = 128


def _cumsum_lanes(x, steps=(1, 2, 4, 8, 16, 32, 64)):
    n = x.shape[1]
    for k in steps:
        pad = jnp.zeros((x.shape[0], k), x.dtype)
        x = x + jnp.concatenate([pad, x[:, : n - k]], axis=1)
    return x


def _cumsum_rows(x):
    n = x.shape[0]
    k = 1
    while k < n:
        pad = jnp.zeros((k, x.shape[1]), x.dtype)
        x = x + jnp.concatenate([pad, x[: n - k]], axis=0)
        k *= 2
    return x


def _router_body(hs_ref, rwt_ref, hsc_ref, pos_ref, md_ref):
    hs = hs_ref[...]
    logits = jnp.dot(hs, rwt_ref[...], preferred_element_type=jnp.float32)
    lane = lax.broadcasted_iota(jnp.int32, (T, LN), 1)
    logits = jnp.where(lane < E, logits, jnp.float32(-1e30))

    mx = jnp.max(logits, axis=1, keepdims=True)
    eq = (logits == mx).astype(jnp.int32)
    first = _cumsum_lanes(eq)
    onehot = jnp.where((eq == 1) & (first == 1), 1, 0)

    score = 1.0 / (1.0 + jnp.exp(-mx))
    hsc_ref[...] = (hs * score).reshape(T * RS, LN)

    ranks = _cumsum_rows(onehot)
    counts = ranks[T - 1 : T, :]
    ptiles = lax.shift_right_logical(counts + (BM - 1), 8)
    s_incl = _cumsum_lanes(ptiles)
    s_excl = s_incl - ptiles
    n_act = s_incl[0:1, LN - 1 : LN]
    poff = s_excl * BM

    pos = (
        jnp.sum(onehot * jnp.broadcast_to(poff, (T, LN)), axis=1, keepdims=True)
        + jnp.sum(onehot * ranks, axis=1, keepdims=True)
        - 1
    )
    sub = lax.broadcasted_iota(jnp.int32, (T, LN), 1)
    pos_ref[...] = (pos * RS + sub)[:, :RS]

    lane_r = lax.broadcasted_iota(jnp.int32, (1, LN), 1)
    veff_row = jnp.minimum(lane_r, n_act - 1)
    actv_row = (lane_r < n_act).astype(jnp.int32)

    w_sub = lax.broadcasted_iota(jnp.int32, (LN, LN), 0)
    lane2 = lax.broadcasted_iota(jnp.int32, (LN, LN), 1)
    veff_sub = jnp.minimum(w_sub, n_act - 1)
    ind = jnp.where(
        (jnp.broadcast_to(s_excl, (LN, LN)) <= veff_sub) & (lane2 < E), 1, 0
    )
    exp_col = jnp.sum(ind, axis=1, keepdims=True) - 1
    ident = (w_sub == lane2).astype(jnp.int32)
    exp_row = jnp.sum(
        jnp.broadcast_to(exp_col, (LN, LN)) * ident, axis=0, keepdims=True
    )
    md_ref[...] = jnp.concatenate(
        [veff_row, exp_row, actv_row, jnp.zeros((5, LN), jnp.int32)], axis=0
    )


def _run_router(hs, rwt, *, interpret=False):
    return pl.pallas_call(
        _router_body,
        out_shape=[
            jax.ShapeDtypeStruct((T * RS, LN), jnp.float32),
            jax.ShapeDtypeStruct((T, RS), jnp.int32),
            jax.ShapeDtypeStruct((8, LN), jnp.int32),
        ],
        interpret=interpret,
    )(hs, rwt)


def _gmm_body(ve_ref, ex_ref, ac_ref, x_ref, gw_ref, uw_ref, dw_ref, out_ref):
    g = pl.program_id(0)

    @pl.when(ac_ref[g] == 1)
    def _():
        xb = x_ref[...].reshape(BM, H).astype(jnp.bfloat16)
        gg = jnp.dot(
            xb, gw_ref[0].astype(jnp.bfloat16), preferred_element_type=jnp.float32
        )
        uu = jnp.dot(
            xb, uw_ref[0].astype(jnp.bfloat16), preferred_element_type=jnp.float32
        )
        act = uu * (gg / (1.0 + jnp.exp(-gg)))
        res = jnp.dot(
            act.astype(jnp.bfloat16),
            dw_ref[0].astype(jnp.bfloat16),
            preferred_element_type=jnp.float32,
        )
        out_ref[...] = res.reshape(BM * RS, LN)


def _run_gmm(ve, ex, ac, xs, gw, uw, dw, *, interpret=False):
    grid_spec = pltpu.PrefetchScalarGridSpec(
        num_scalar_prefetch=3,
        grid=(NV,),
        in_specs=[
            pl.BlockSpec((BM * RS, LN), lambda g, ve, ex, ac: (ve[g], 0)),
            pl.BlockSpec((1, H, I2), lambda g, ve, ex, ac: (ex[g], 0, 0)),
            pl.BlockSpec((1, H, I2), lambda g, ve, ex, ac: (ex[g], 0, 1)),
            pl.BlockSpec((1, I2, H), lambda g, ve, ex, ac: (ex[g], 0, 0)),
        ],
        out_specs=pl.BlockSpec((BM * RS, LN), lambda g, ve, ex, ac: (ve[g], 0)),
    )
    return pl.pallas_call(
        _gmm_body,
        grid_spec=grid_spec,
        out_shape=jax.ShapeDtypeStruct((TPAD * RS, LN), jnp.float32),
        interpret=interpret,
    )(ve, ex, ac, xs, gw, uw, dw)


def _tn_dot(a, b_t):
    return lax.dot_general(
        a, b_t, (((1,), (1,)), ((), ())), preferred_element_type=jnp.float32
    )


def _shared_body(hs_ref, y_ref, gt_ref, ut_ref, dt_ref, o_ref):
    xb = hs_ref[...].astype(jnp.bfloat16)
    gg = _tn_dot(xb, gt_ref[...].astype(jnp.bfloat16))
    uu = _tn_dot(xb, ut_ref[...].astype(jnp.bfloat16))
    act = (gg / (1.0 + jnp.exp(-gg))) * uu
    o_ref[...] = _tn_dot(
        act.astype(jnp.bfloat16), dt_ref[...].astype(jnp.bfloat16)
    ) + y_ref[...].reshape(BM, H)


def _run_shared(hs, y, gt, ut, dt, *, interpret=False):
    nb = T // BM
    return pl.pallas_call(
        _shared_body,
        grid=(nb,),
        in_specs=[
            pl.BlockSpec((BM, H), lambda i: (i, 0)),
            pl.BlockSpec((BM * RS, LN), lambda i: (i, 0)),
            pl.BlockSpec((I2, H), lambda i: (0, 0)),
            pl.BlockSpec((I2, H), lambda i: (0, 0)),
            pl.BlockSpec((H, I2), lambda i: (0, 0)),
        ],
        out_specs=pl.BlockSpec((BM, H), lambda i: (i, 0)),
        out_shape=jax.ShapeDtypeStruct((T, H), jnp.float32),
        interpret=interpret,
    )(hs, y, gt, ut, dt)


def _sc_mesh():
    return plsc.VectorSubcoreMesh(core_axis_name="core", subcore_axis_name="subcore")


def _sc_scatter_rows(rows, idx2d):
    nsub = T * RS

    @functools.partial(
        pl.kernel,
        out_type=jax.ShapeDtypeStruct((TPAD * RS, LN), jnp.float32),
        mesh=_sc_mesh(),
    )
    def k(x_hbm, i_hbm, o_hbm):
        def body(x_vmem, i_vmem):
            pltpu.sync_copy(x_vmem, o_hbm.at[i_vmem.at[0]])

        pltpu.emit_pipeline(
            body,
            grid=(nsub // GW,),
            in_specs=[
                pl.BlockSpec((GW, LN), lambda i: (i, 0)),
                pl.BlockSpec((1, GW), lambda i: (0, i)),
            ],
            out_specs=[],
            core_axis_name="subcore",
            dimension_semantics=(pltpu.PARALLEL,),
        )(x_hbm, i_hbm)

    return k(rows, idx2d)


def _sc_gather_rows(table, idx2d):
    nsub = T * RS

    @functools.partial(
        pl.kernel,
        out_type=jax.ShapeDtypeStruct((nsub, LN), jnp.float32),
        mesh=_sc_mesh(),
    )
    def k(y_hbm, i_hbm, o_hbm):
        def body(i_vmem, o_vmem):
            pltpu.sync_copy(y_hbm.at[i_vmem.at[0]], o_vmem)

        pltpu.emit_pipeline(
            body,
            grid=(nsub // GW,),
            in_specs=[pl.BlockSpec((1, GW), lambda i: (0, i))],
            out_specs=[pl.BlockSpec((GW, LN), lambda i: (i, 0))],
            core_axis_name="subcore",
            dimension_semantics=(pltpu.PARALLEL,),
        )(i_hbm, o_hbm)

    return k(table, idx2d)


def kernel(hidden_states, gate_up_proj, down_proj, router_w,
           shared_gate_w, shared_up_w, shared_down_w):
    hs = hidden_states.reshape(-1, H)
    rwt = jnp.pad(router_w.T, ((0, 0), (0, LN - E)))

    hsc, pos8, md = _run_router(hs, rwt)
    ve, ex, ac = md[0], md[1], md[2]
    pos2d = pos8.reshape(1, T * RS)

    xs = _sc_scatter_rows(hsc, pos2d)
    ys = _run_gmm(ve, ex, ac, xs, gate_up_proj, gate_up_proj, down_proj)
    y = _sc_gather_rows(ys, pos2d)

    return _run_shared(hs, y, shared_gate_w, shared_up_w, shared_down_w)


# speedup vs baseline: 3.6974x; 1.0991x over previous
import functools

import jax
import jax.numpy as jnp
from jax import lax
from jax.experimental import pallas as pl
from jax.experimental.pallas import tpu as pltpu
from jax.experimental.pallas import tpu_sc as plsc

E = 8
H = 1024
I2 = 2048
T = 4096
BM = 256
NV = 23
TPAD = NV * BM
LN = 128
RS = H // LN
GW = 128


def _cumsum_lanes(x, steps=(1, 2, 4, 8, 16, 32, 64)):
    n = x.shape[1]
    for k in steps:
        pad = jnp.zeros((x.shape[0], k), x.dtype)
        x = x + jnp.concatenate([pad, x[:, : n - k]], axis=1)
    return x


def _cumsum_rows(x):
    n = x.shape[0]
    k = 1
    while k < n:
        pad = jnp.zeros((k, x.shape[1]), x.dtype)
        x = x + jnp.concatenate([pad, x[: n - k]], axis=0)
        k *= 2
    return x


def _router_body(hs_ref, rwt_ref, hsc_ref, pos_ref, md_ref):
    hs = hs_ref[...]
    logits = jnp.dot(hs, rwt_ref[...], preferred_element_type=jnp.float32)
    lane = lax.broadcasted_iota(jnp.int32, (T, LN), 1)
    logits = jnp.where(lane < E, logits, jnp.float32(-1e30))

    mx = jnp.max(logits, axis=1, keepdims=True)
    eq = (logits == mx).astype(jnp.int32)
    first = _cumsum_lanes(eq)
    onehot = jnp.where((eq == 1) & (first == 1), 1, 0)

    score = 1.0 / (1.0 + jnp.exp(-mx))
    hsc_ref[...] = (hs * score).reshape(T * RS, LN)

    ranks = _cumsum_rows(onehot)
    counts = ranks[T - 1 : T, :]
    ptiles = lax.shift_right_logical(counts + (BM - 1), 8)
    s_incl = _cumsum_lanes(ptiles)
    s_excl = s_incl - ptiles
    n_act = s_incl[0:1, LN - 1 : LN]
    poff = s_excl * BM

    pos = (
        jnp.sum(onehot * jnp.broadcast_to(poff, (T, LN)), axis=1, keepdims=True)
        + jnp.sum(onehot * ranks, axis=1, keepdims=True)
        - 1
    )
    sub = lax.broadcasted_iota(jnp.int32, (T, LN), 1)
    pos_ref[...] = (pos * RS + sub)[:, :RS]

    lane_r = lax.broadcasted_iota(jnp.int32, (1, LN), 1)
    veff_row = jnp.minimum(lane_r, n_act - 1)
    actv_row = (lane_r < n_act).astype(jnp.int32)

    w_sub = lax.broadcasted_iota(jnp.int32, (LN, LN), 0)
    lane2 = lax.broadcasted_iota(jnp.int32, (LN, LN), 1)
    veff_sub = jnp.minimum(w_sub, n_act - 1)
    ind = jnp.where(
        (jnp.broadcast_to(s_excl, (LN, LN)) <= veff_sub) & (lane2 < E), 1, 0
    )
    exp_col = jnp.sum(ind, axis=1, keepdims=True) - 1
    ident = (w_sub == lane2).astype(jnp.int32)
    exp_row = jnp.sum(
        jnp.broadcast_to(exp_col, (LN, LN)) * ident, axis=0, keepdims=True
    )
    md_ref[...] = jnp.concatenate(
        [veff_row, exp_row, actv_row, jnp.zeros((5, LN), jnp.int32)], axis=0
    )


def _run_router(hs, rwt, *, interpret=False):
    return pl.pallas_call(
        _router_body,
        out_shape=[
            jax.ShapeDtypeStruct((T * RS, LN), jnp.float32),
            jax.ShapeDtypeStruct((T, RS), jnp.int32),
            jax.ShapeDtypeStruct((8, LN), jnp.int32),
        ],
        interpret=interpret,
    )(hs, rwt)


def _gmm_body(ve_ref, ex_ref, ac_ref, x_ref, gw_ref, uw_ref, dw_ref, out_ref):
    g = pl.program_id(0)

    @pl.when(ac_ref[g] == 1)
    def _():
        xb = x_ref[...].reshape(BM, H).astype(jnp.bfloat16)
        gg = jnp.dot(
            xb, gw_ref[0].astype(jnp.bfloat16), preferred_element_type=jnp.float32
        )
        uu = jnp.dot(
            xb, uw_ref[0].astype(jnp.bfloat16), preferred_element_type=jnp.float32
        )
        act = uu * (gg / (1.0 + jnp.exp(-gg)))
        res = jnp.dot(
            act.astype(jnp.bfloat16),
            dw_ref[0].astype(jnp.bfloat16),
            preferred_element_type=jnp.float32,
        )
        out_ref[...] = res.reshape(BM * RS, LN)


def _run_gmm(ve, ex, ac, xs, gw, uw, dw, *, interpret=False):
    grid_spec = pltpu.PrefetchScalarGridSpec(
        num_scalar_prefetch=3,
        grid=(NV,),
        in_specs=[
            pl.BlockSpec((BM * RS, LN), lambda g, ve, ex, ac: (ve[g], 0)),
            pl.BlockSpec((1, H, I2), lambda g, ve, ex, ac: (ex[g], 0, 0)),
            pl.BlockSpec((1, H, I2), lambda g, ve, ex, ac: (ex[g], 0, 1)),
            pl.BlockSpec((1, I2, H), lambda g, ve, ex, ac: (ex[g], 0, 0)),
        ],
        out_specs=pl.BlockSpec((BM * RS, LN), lambda g, ve, ex, ac: (ve[g], 0)),
    )
    return pl.pallas_call(
        _gmm_body,
        grid_spec=grid_spec,
        out_shape=jax.ShapeDtypeStruct((TPAD * RS, LN), jnp.float32),
        interpret=interpret,
    )(ve, ex, ac, xs, gw, uw, dw)


def _tn_dot(a, b_t):
    return lax.dot_general(
        a, b_t, (((1,), (1,)), ((), ())), preferred_element_type=jnp.float32
    )


def _shared_body(hs_ref, y_ref, gt_ref, ut_ref, dt_ref, o_ref):
    xb = hs_ref[...].astype(jnp.bfloat16)
    gg = _tn_dot(xb, gt_ref[...].astype(jnp.bfloat16))
    uu = _tn_dot(xb, ut_ref[...].astype(jnp.bfloat16))
    act = (gg / (1.0 + jnp.exp(-gg))) * uu
    o_ref[...] = _tn_dot(
        act.astype(jnp.bfloat16), dt_ref[...].astype(jnp.bfloat16)
    ) + y_ref[...].reshape(BM, H)


def _run_shared(hs, y, gt, ut, dt, *, interpret=False):
    nb = T // BM
    return pl.pallas_call(
        _shared_body,
        grid=(nb,),
        in_specs=[
            pl.BlockSpec((BM, H), lambda i: (i, 0)),
            pl.BlockSpec((BM * RS, LN), lambda i: (i, 0)),
            pl.BlockSpec((I2, H), lambda i: (0, 0)),
            pl.BlockSpec((I2, H), lambda i: (0, 0)),
            pl.BlockSpec((H, I2), lambda i: (0, 0)),
        ],
        out_specs=pl.BlockSpec((BM, H), lambda i: (i, 0)),
        out_shape=jax.ShapeDtypeStruct((T, H), jnp.float32),
        interpret=interpret,
    )(hs, y, gt, ut, dt)


def _sc_mesh():
    return plsc.VectorSubcoreMesh(core_axis_name="core", subcore_axis_name="subcore")


def _sc_scatter_rows(rows, idx2d):
    nsub = T * RS

    @functools.partial(
        pl.kernel,
        out_type=jax.ShapeDtypeStruct((TPAD * RS, LN), jnp.float32),
        mesh=_sc_mesh(),
    )
    def k(x_hbm, i_hbm, o_hbm):
        def body(x_vmem, i_vmem):
            pltpu.sync_copy(x_vmem, o_hbm.at[i_vmem.at[0]])

        pltpu.emit_pipeline(
            body,
            grid=(nsub // GW,),
            in_specs=[
                pl.BlockSpec((GW, LN), lambda i: (i, 0)),
                pl.BlockSpec((1, GW), lambda i: (0, i)),
            ],
            out_specs=[],
            core_axis_name=("core", "subcore"),
            dimension_semantics=(pltpu.PARALLEL,),
        )(x_hbm, i_hbm)

    return k(rows, idx2d)


def _sc_gather_rows(table, idx2d):
    nsub = T * RS

    @functools.partial(
        pl.kernel,
        out_type=jax.ShapeDtypeStruct((nsub, LN), jnp.float32),
        mesh=_sc_mesh(),
    )
    def k(y_hbm, i_hbm, o_hbm):
        def body(i_vmem, o_vmem):
            pltpu.sync_copy(y_hbm.at[i_vmem.at[0]], o_vmem)

        pltpu.emit_pipeline(
            body,
            grid=(nsub // GW,),
            in_specs=[pl.BlockSpec((1, GW), lambda i: (0, i))],
            out_specs=[pl.BlockSpec((GW, LN), lambda i: (i, 0))],
            core_axis_name=("core", "subcore"),
            dimension_semantics=(pltpu.PARALLEL,),
        )(i_hbm, o_hbm)

    return k(table, idx2d)


def kernel(hidden_states, gate_up_proj, down_proj, router_w,
           shared_gate_w, shared_up_w, shared_down_w):
    hs = hidden_states.reshape(-1, H)
    rwt = jnp.pad(router_w.T, ((0, 0), (0, LN - E)))

    hsc, pos8, md = _run_router(hs, rwt)
    ve, ex, ac = md[0], md[1], md[2]
    pos2d = pos8.reshape(1, T * RS)

    xs = _sc_scatter_rows(hsc, pos2d)
    ys = _run_gmm(ve, ex, ac, xs, gate_up_proj, gate_up_proj, down_proj)
    y = _sc_gather_rows(ys, pos2d)

    return _run_shared(hs, y, shared_gate_w, shared_up_w, shared_down_w)
